# baseline (speedup 1.0000x reference)
import functools

import jax
import jax.numpy as jnp
from jax import lax
from jax.experimental import pallas as pl
from jax.experimental.pallas import tpu as pltpu
from jax.experimental.pallas import tpu_sc as plsc

B = 4096
S = 200
D = 64
NC = 2
NS = 16
NW = NC * NS
RPW = B // NW
T_STEADY = 42

_mesh = plsc.VectorSubcoreMesh(core_axis_name="c", subcore_axis_name="s")


@functools.partial(
    pl.kernel,
    mesh=_mesh,
    out_type=jax.ShapeDtypeStruct((B, S, D), jnp.float32),
    scratch_types=[
        pltpu.VMEM((RPW, S), jnp.int32),
        pltpu.VMEM((S, D), jnp.float32),
        pltpu.VMEM((S, D), jnp.float32),
        pltpu.VMEM((S, D), jnp.float32),
        pltpu.SemaphoreType.DMA,
        pltpu.SemaphoreType.DMA,
        pltpu.SemaphoreType.DMA,
        pltpu.SemaphoreType.DMA,
        pltpu.SemaphoreType.DMA,
        pltpu.SemaphoreType.DMA,
    ],
    compiler_params=pltpu.CompilerParams(use_tc_tiling_on_sc=True),
)
def _emb_lookup(idx_hbm, table_hbm, out_hbm, idx_all,
                rows0, rows1, rows2, g0, g1, g2, s0, s1, s2):
    rows = (rows0, rows1, rows2)
    gsem = (g0, g1, g2)
    ssem = (s0, s1, s2)
    wid = lax.axis_index("s") * NC + lax.axis_index("c")
    base = wid * RPW

    def enq(vec, j, pos, lanes):
        for l in lanes:
            pltpu.async_copy(table_hbm.at[vec[l]], rows[j].at[pos + l],
                             gsem[j])

    def fire(g, j):
        for c in range(12):
            enq(idx_all[g, pl.ds(c * 16, 16)], j, c * 16, range(16))
        enq(idx_all[g, pl.ds(S - 16, 16)], j, S - 16, range(8, 16))

    def fire_small(g, j):
        def chunk(c, carry):
            enq(idx_all[g, pl.ds(c * 16, 16)], j, c * 16, range(16))
            return carry

        lax.fori_loop(0, 12, chunk, 0)
        enq(idx_all[g, pl.ds(S - 16, 16)], j, S - 16, range(8, 16))

    def wait_fire(j):
        pltpu.make_async_copy(out_hbm.at[0], rows[j], gsem[j]).wait()

    def store(g, j):
        pltpu.async_copy(rows[j], out_hbm.at[base + g], ssem[j])

    def wait_store(j):
        pltpu.make_async_copy(rows[j], out_hbm.at[0], ssem[j]).wait()

    pltpu.sync_copy(idx_hbm.at[pl.ds(base, RPW)], idx_all)

    fire_small(0, 0)
    fire_small(1, 1)
    fire_small(2, 2)
    wait_fire(0)
    store(0, 0)
    wait_fire(1)
    store(1, 1)

    def body(t, carry):
        for j in range(3):
            g = 3 * t + j
            p = (j + 2) % 3
            wait_store(j)
            fire(g, j)
            wait_fire(p)
            store(g - 1, p)
        return carry

    lax.fori_loop(1, T_STEADY, body, 0)

    wait_store(0)
    fire_small(126, 0)
    wait_fire(2)
    store(125, 2)
    wait_store(1)
    fire_small(127, 1)
    wait_fire(0)
    store(126, 0)
    wait_fire(1)
    store(127, 1)
    wait_store(0)
    wait_store(1)
    wait_store(2)


def kernel(input_ids, weight):
    return _emb_lookup(input_ids.astype(jnp.int32), weight)


# speedup vs baseline: 1.0785x; 1.0785x over previous
import functools

import jax
import jax.numpy as jnp
from jax import lax
from jax.experimental import pallas as pl
from jax.experimental.pallas import tpu as pltpu
from jax.experimental.pallas import tpu_sc as plsc

B = 4096
S = 200
D = 64
DP = 128
V = 1000000
NC = 2
NS = 16
NW = NC * NS
RPW = B // NW
T_STEADY = 42

_mesh = plsc.VectorSubcoreMesh(core_axis_name="c", subcore_axis_name="s")


@functools.partial(
    pl.kernel,
    mesh=_mesh,
    out_type=jax.ShapeDtypeStruct((B, S, DP), jnp.float32),
    scratch_types=[
        pltpu.VMEM((RPW, S), jnp.int32),
        pltpu.VMEM((S, D), jnp.float32),
        pltpu.VMEM((S, D), jnp.float32),
        pltpu.VMEM((S, D), jnp.float32),
        pltpu.SemaphoreType.DMA,
        pltpu.SemaphoreType.DMA,
        pltpu.SemaphoreType.DMA,
        pltpu.SemaphoreType.DMA,
        pltpu.SemaphoreType.DMA,
        pltpu.SemaphoreType.DMA,
    ],
    compiler_params=pltpu.CompilerParams(use_tc_tiling_on_sc=False),
)
def _emb_lookup(idx_hbm, table_hbm, out_hbm, idx_all, rows0, rows1, rows2,
                g0, g1, g2, s0, s1, s2):
    rows = (rows0, rows1, rows2)
    gsem = (g0, g1, g2)
    ssem = (s0, s1, s2)
    wid = lax.axis_index("s") * NC + lax.axis_index("c")
    base = wid * RPW

    def fire(g, j):
        pltpu.async_copy(table_hbm.at[idx_all.at[g, pl.ds(0, 128)]],
                         rows[j].at[pl.ds(0, 128)], gsem[j])
        pltpu.async_copy(table_hbm.at[idx_all.at[g, pl.ds(128, S - 128)]],
                         rows[j].at[pl.ds(128, S - 128)], gsem[j])

    def wait_fire(j):
        pltpu.make_async_copy(out_hbm.at[0, :, pl.ds(0, D)], rows[j],
                              gsem[j]).wait()

    def store(g, j):
        pltpu.async_copy(rows[j], out_hbm.at[base + g, :, pl.ds(0, D)],
                         ssem[j])

    def wait_store(j):
        pltpu.make_async_copy(rows[j], out_hbm.at[0, :, pl.ds(0, D)],
                              ssem[j]).wait()

    pltpu.sync_copy(idx_hbm.at[pl.ds(base, RPW)], idx_all)

    fire(0, 0)
    fire(1, 1)
    fire(2, 2)
    wait_fire(0)
    store(0, 0)
    wait_fire(1)
    store(1, 1)

    def body(t, carry):
        for j in range(3):
            g = 3 * t + j
            p = (j + 2) % 3
            wait_store(j)
            fire(g, j)
            wait_fire(p)
            store(g - 1, p)
        return carry

    lax.fori_loop(1, T_STEADY, body, 0)

    wait_store(0)
    fire(126, 0)
    wait_fire(2)
    store(125, 2)
    wait_store(1)
    fire(127, 1)
    wait_fire(0)
    store(126, 0)
    wait_fire(1)
    store(127, 1)
    wait_store(0)
    wait_store(1)
    wait_store(2)


def kernel(input_ids, weight):
    wt = jnp.pad(weight, ((0, 0), (0, DP - D))).reshape(2 * V, D)
    ids2 = input_ids.astype(jnp.int32) * 2
    out = _emb_lookup(ids2, wt)
    return out[..., :D]
